# baseline (speedup 1.0000x reference)
import functools
import jax
import jax.numpy as jnp
from jax import lax
from jax.experimental import pallas as pl
from jax.experimental.pallas import tpu as pltpu
from jax.experimental.pallas import tpu_sc as plsc

_NW = 32
_LANE = 16


def _make_detile(vocab, emb):
    n_vt = vocab // 128
    tail = vocab - n_vt * 128
    per_w = (n_vt + _NW - 1) // _NW
    mesh = plsc.VectorSubcoreMesh(core_axis_name="c", subcore_axis_name="s")

    @functools.partial(
        pl.kernel,
        mesh=mesh,
        out_type=jax.ShapeDtypeStruct((vocab * emb,), jnp.float32),
        scratch_types=[
            pltpu.VMEM((64, 128), jnp.float32),
            pltpu.VMEM((3 * 4096,), jnp.float32),
            pltpu.VMEM((tail * emb,), jnp.float32),
            pltpu.SemaphoreType.DMA((2,)),
            pltpu.SemaphoreType.DMA((3,)),
        ],
        compiler_params=pltpu.CompilerParams(
            use_tc_tiling_on_sc=True, needs_layout_passes=False
        ),
    )
    def detile_kernel(table_t, tail_rm, tr_out, vin, vout, vtail, lsem, osem):
        wid = lax.axis_index("s") * 2 + lax.axis_index("c")
        lanes = lax.iota(jnp.int32, _LANE)

        def fire_loads(vt, p):
            for eh in range(4):
                pltpu.async_copy(
                    table_t.at[pl.ds(eh * 8, 8), pl.ds(vt * 128, 128)],
                    vin.at[pl.ds(p * 32 + eh * 8, 8)],
                    lsem.at[p],
                )

        def wait_loads(p):
            pltpu.make_async_copy(
                table_t.at[pl.ds(0, 32), pl.ds(0, 128)],
                vin.at[pl.ds(p * 32, 32)],
                lsem.at[p],
            ).wait()

        def wait_out(s):
            pltpu.make_async_copy(
                vout.at[pl.ds(s * 4096, 4096)],
                tr_out.at[pl.ds(0, 4096)],
                osem.at[s],
            ).wait()

        def transpose_and_write(vt, p, s):
            @pl.loop(0, 128)
            def _(vl):
                vlv = lanes * 0 + vl
                for e0 in (0, 16):
                    vals = plsc.load_gather(vin, [lanes + (p * 32 + e0), vlv])
                    vout[pl.ds(s * 4096 + vl * 32 + e0, _LANE)] = vals

            pltpu.async_copy(
                vout.at[pl.ds(s * 4096, 4096)],
                tr_out.at[pl.ds(vt * 4096, 4096)],
                osem.at[s],
            )

        fire_loads(wid, 0)

        @pl.loop(0, per_w)
        def _(i):
            vt = i * _NW + wid
            vt_next = vt + _NW
            p = lax.rem(i, 2)
            s = lax.rem(i, 3)

            @pl.when(vt_next < n_vt)
            def _():
                fire_loads(vt_next, 1 - p)

            @pl.when(vt < n_vt)
            def _():
                wait_loads(p)

                @pl.when(i >= 3)
                def _():
                    wait_out(s)

                transpose_and_write(vt, p, s)

        for d in range(max(0, per_w - 4), per_w):
            vt_d = d * _NW + wid

            @pl.when((vt_d < n_vt) & (vt_d + 3 * _NW >= n_vt))
            def _():
                wait_out(d % 3)

        @pl.when(wid == 0)
        def _():
            pltpu.sync_copy(tail_rm, vtail)
            pltpu.sync_copy(vtail, tr_out.at[pl.ds(n_vt * 4096, tail * emb)])

    return detile_kernel


def _make_gather(n_rows, vocab, emb):
    rows_per_w = n_rows // _NW
    chunk = 8
    n_chunks = rows_per_w // chunk
    unit = 128 * emb
    mesh = plsc.VectorSubcoreMesh(core_axis_name="c", subcore_axis_name="s")

    @functools.partial(
        pl.kernel,
        mesh=mesh,
        out_type=jax.ShapeDtypeStruct((200 * 131072,), jnp.float32),
        scratch_types=[
            pltpu.VMEM((2 * chunk, 128), jnp.int32),
            pltpu.VMEM((2 * chunk * 128, emb), jnp.float32),
            pltpu.VMEM((4 * unit,), jnp.float32),
            pltpu.SemaphoreType.DMA((2,)),
            pltpu.SemaphoreType.DMA((4,)),
        ],
        compiler_params=pltpu.CompilerParams(
            use_tc_tiling_on_sc=False, needs_layout_passes=False
        ),
    )
    def gather_kernel(idx_hbm, table_rm, out_hbm, idx_v, rows_v, stg, gsem, osem):
        wid = lax.axis_index("s") * 2 + lax.axis_index("c")
        base_row = wid * rows_per_w
        lanes = lax.iota(jnp.int32, _LANE)

        def fire_chunk(c, p):
            r = base_row + c * chunk
            pltpu.sync_copy(
                idx_hbm.at[pl.ds(r, chunk)], idx_v.at[pl.ds(p * chunk, chunk)]
            )
            for j in range(chunk):
                pltpu.async_copy(
                    table_rm.at[idx_v.at[p * chunk + j]],
                    rows_v.at[pl.ds((p * chunk + j) * 128, 128)],
                    gsem.at[p],
                )

        def wait_gathers(p):
            pltpu.make_async_copy(
                table_rm.at[pl.ds(0, chunk * 128)],
                rows_v.at[pl.ds(p * chunk * 128, chunk * 128)],
                gsem.at[p],
            ).wait()

        def emit_chunk(ec, p):
            r = base_row + ec * chunk

            @pl.loop(0, chunk)
            def _(j):
                rr = r + j
                l = lax.div(rr, 32)
                bj = lax.rem(rr, 32)
                gu = ec * chunk + j
                sp = lax.rem(gu, 4)

                @pl.when(gu >= 4)
                def _():
                    pltpu.make_async_copy(
                        stg.at[pl.ds(0, unit)],
                        out_hbm.at[pl.ds(0, unit)],
                        osem.at[sp],
                    ).wait()

                src0 = (p * chunk + j) * 128

                @pl.loop(0, emb)
                def _(e):
                    ev = lanes * 0 + e
                    for b0 in range(0, 128, _LANE):
                        vals = plsc.load_gather(
                            rows_v, [lanes + (src0 + b0), ev]
                        )
                        stg[pl.ds(sp * unit + e * 128 + b0, _LANE)] = vals

                for eh in range(4):
                    pltpu.async_copy(
                        stg.at[pl.ds(sp * unit + eh * 1024, 1024)],
                        out_hbm.at[
                            pl.ds(l * 131072 + eh * 32 * 1024 + bj * 1024, 1024)
                        ],
                        osem.at[sp],
                    )

        fire_chunk(0, 0)

        @pl.loop(0, (n_chunks - 1) // 2)
        def _(g):
            for t in range(2):
                c = 1 + g * 2 + t
                p = (1 + t) % 2
                fire_chunk(c, p)
                wait_gathers(1 - p)
                emit_chunk(c - 1, 1 - p)

        p_last = (n_chunks - 1) % 2
        wait_gathers(p_last)
        emit_chunk(n_chunks - 1, p_last)
        for sp in range(4):
            pltpu.make_async_copy(
                stg.at[pl.ds(0, unit)], out_hbm.at[pl.ds(0, unit)], osem.at[sp]
            ).wait()

    return gather_kernel


def kernel(x, table):
    b, l = x.shape
    vocab, emb = table.shape
    n = b * l

    table_t = table.T
    n_vt = vocab // 128
    tail_rm = table[n_vt * 128:, :].reshape(-1)
    tr = _make_detile(vocab, emb)(table_t, tail_rm)
    table_rm = tr.reshape(vocab, emb)

    idx = x.T.reshape(n // 128, 128)
    out5 = _make_gather(n // 128, vocab, emb)(idx, table_rm)

    out = (
        out5.reshape(l, 4, 32, 8, 128)
        .transpose(2, 4, 0, 1, 3)
        .reshape(b, l, emb)
    )
    return out


# speedup vs baseline: 1.0002x; 1.0002x over previous
import functools
import jax
import jax.numpy as jnp
from jax import lax
from jax.experimental import pallas as pl
from jax.experimental.pallas import tpu as pltpu
from jax.experimental.pallas import tpu_sc as plsc

_NW = 32
_LANE = 16


def _make_detile(vocab, emb):
    n_vt = vocab // 128
    tail = vocab - n_vt * 128
    per_w = (n_vt + _NW - 1) // _NW
    mesh = plsc.VectorSubcoreMesh(core_axis_name="c", subcore_axis_name="s")

    @functools.partial(
        pl.kernel,
        mesh=mesh,
        out_type=jax.ShapeDtypeStruct((vocab * emb,), jnp.float32),
        scratch_types=[
            pltpu.VMEM((64, 128), jnp.float32),
            pltpu.VMEM((3 * 4096,), jnp.float32),
            pltpu.VMEM((tail * emb,), jnp.float32),
            pltpu.SemaphoreType.DMA((2,)),
            pltpu.SemaphoreType.DMA((3,)),
        ],
        compiler_params=pltpu.CompilerParams(
            use_tc_tiling_on_sc=True, needs_layout_passes=False
        ),
    )
    def detile_kernel(table_t, tail_rm, tr_out, vin, vout, vtail, lsem, osem):
        wid = lax.axis_index("s") * 2 + lax.axis_index("c")
        lanes = lax.iota(jnp.int32, _LANE)

        def fire_loads(vt, p):
            for eh in range(4):
                pltpu.async_copy(
                    table_t.at[pl.ds(eh * 8, 8), pl.ds(vt * 128, 128)],
                    vin.at[pl.ds(p * 32 + eh * 8, 8)],
                    lsem.at[p],
                )

        def wait_loads(p):
            pltpu.make_async_copy(
                table_t.at[pl.ds(0, 32), pl.ds(0, 128)],
                vin.at[pl.ds(p * 32, 32)],
                lsem.at[p],
            ).wait()

        def wait_out(s):
            pltpu.make_async_copy(
                vout.at[pl.ds(s * 4096, 4096)],
                tr_out.at[pl.ds(0, 4096)],
                osem.at[s],
            ).wait()

        def transpose_and_write(vt, p, s):
            row0 = lanes + p * 32
            row1 = row0 + 16
            so = s * 4096
            for vl in range(128):
                vlv = lanes * 0 + vl
                vout[pl.ds(so + vl * 32, _LANE)] = plsc.load_gather(
                    vin, [row0, vlv]
                )
                vout[pl.ds(so + vl * 32 + 16, _LANE)] = plsc.load_gather(
                    vin, [row1, vlv]
                )

            pltpu.async_copy(
                vout.at[pl.ds(s * 4096, 4096)],
                tr_out.at[pl.ds(vt * 4096, 4096)],
                osem.at[s],
            )

        fire_loads(wid, 0)

        @pl.loop(0, per_w)
        def _(i):
            vt = i * _NW + wid
            vt_next = vt + _NW
            p = lax.rem(i, 2)
            s = lax.rem(i, 3)

            @pl.when(vt_next < n_vt)
            def _():
                fire_loads(vt_next, 1 - p)

            @pl.when(vt < n_vt)
            def _():
                wait_loads(p)

                @pl.when(i >= 3)
                def _():
                    wait_out(s)

                transpose_and_write(vt, p, s)

        for d in range(max(0, per_w - 4), per_w):
            vt_d = d * _NW + wid

            @pl.when((vt_d < n_vt) & (vt_d + 3 * _NW >= n_vt))
            def _():
                wait_out(d % 3)

        @pl.when(wid == 0)
        def _():
            pltpu.sync_copy(tail_rm, vtail)
            pltpu.sync_copy(vtail, tr_out.at[pl.ds(n_vt * 4096, tail * emb)])

    return detile_kernel


def _make_gather(n_rows, vocab, emb):
    rows_per_w = n_rows // _NW
    chunk = 8
    n_chunks = rows_per_w // chunk
    unit = 128 * emb
    mesh = plsc.VectorSubcoreMesh(core_axis_name="c", subcore_axis_name="s")

    @functools.partial(
        pl.kernel,
        mesh=mesh,
        out_type=jax.ShapeDtypeStruct((200 * 131072,), jnp.float32),
        scratch_types=[
            pltpu.VMEM((2 * chunk, 128), jnp.int32),
            pltpu.VMEM((2 * chunk * 128, emb), jnp.float32),
            pltpu.VMEM((4 * unit,), jnp.float32),
            pltpu.SemaphoreType.DMA((2,)),
            pltpu.SemaphoreType.DMA((4,)),
        ],
        compiler_params=pltpu.CompilerParams(
            use_tc_tiling_on_sc=False, needs_layout_passes=False
        ),
    )
    def gather_kernel(idx_hbm, table_rm, out_hbm, idx_v, rows_v, stg, gsem, osem):
        wid = lax.axis_index("s") * 2 + lax.axis_index("c")
        base_row = wid * rows_per_w
        lanes = lax.iota(jnp.int32, _LANE)

        def fire_chunk(c, p):
            r = base_row + c * chunk
            pltpu.sync_copy(
                idx_hbm.at[pl.ds(r, chunk)], idx_v.at[pl.ds(p * chunk, chunk)]
            )
            for j in range(chunk):
                pltpu.async_copy(
                    table_rm.at[idx_v.at[p * chunk + j]],
                    rows_v.at[pl.ds((p * chunk + j) * 128, 128)],
                    gsem.at[p],
                )

        def wait_gathers(p):
            pltpu.make_async_copy(
                table_rm.at[pl.ds(0, chunk * 128)],
                rows_v.at[pl.ds(p * chunk * 128, chunk * 128)],
                gsem.at[p],
            ).wait()

        def emit_chunk(ec, p):
            r = base_row + ec * chunk

            @pl.loop(0, chunk)
            def _(j):
                rr = r + j
                l = lax.div(rr, 32)
                bj = lax.rem(rr, 32)
                gu = ec * chunk + j
                sp = lax.rem(gu, 4)

                @pl.when(gu >= 4)
                def _():
                    pltpu.make_async_copy(
                        stg.at[pl.ds(0, unit)],
                        out_hbm.at[pl.ds(0, unit)],
                        osem.at[sp],
                    ).wait()

                src0 = (p * chunk + j) * 128
                sbase = sp * unit
                rowv = [lanes + (src0 + b0) for b0 in range(0, 128, _LANE)]
                for e in range(emb):
                    ev = lanes * 0 + e
                    for bi in range(8):
                        vals = plsc.load_gather(rows_v, [rowv[bi], ev])
                        stg[pl.ds(sbase + e * 128 + bi * _LANE, _LANE)] = vals

                for eh in range(4):
                    pltpu.async_copy(
                        stg.at[pl.ds(sp * unit + eh * 1024, 1024)],
                        out_hbm.at[
                            pl.ds(l * 131072 + eh * 32 * 1024 + bj * 1024, 1024)
                        ],
                        osem.at[sp],
                    )

        fire_chunk(0, 0)

        @pl.loop(0, (n_chunks - 1) // 2)
        def _(g):
            for t in range(2):
                c = 1 + g * 2 + t
                p = (1 + t) % 2
                fire_chunk(c, p)
                wait_gathers(1 - p)
                emit_chunk(c - 1, 1 - p)

        p_last = (n_chunks - 1) % 2
        wait_gathers(p_last)
        emit_chunk(n_chunks - 1, p_last)
        for sp in range(4):
            pltpu.make_async_copy(
                stg.at[pl.ds(0, unit)], out_hbm.at[pl.ds(0, unit)], osem.at[sp]
            ).wait()

    return gather_kernel


def kernel(x, table):
    b, l = x.shape
    vocab, emb = table.shape
    n = b * l

    table_t = table.T
    n_vt = vocab // 128
    tail_rm = table[n_vt * 128:, :].reshape(-1)
    tr = _make_detile(vocab, emb)(table_t, tail_rm)
    table_rm = tr.reshape(vocab, emb)

    idx = x.T.reshape(n // 128, 128)
    out5 = _make_gather(n // 128, vocab, emb)(idx, table_rm)

    out = (
        out5.reshape(l, 4, 32, 8, 128)
        .transpose(2, 4, 0, 1, 3)
        .reshape(b, l, emb)
    )
    return out


# speedup vs baseline: 1.8418x; 1.8414x over previous
import functools
import jax
import jax.numpy as jnp
from jax import lax
from jax.experimental import pallas as pl
from jax.experimental.pallas import tpu as pltpu
from jax.experimental.pallas import tpu_sc as plsc

_NW = 32
_LANE = 16


def _make_detile(vocab, emb):
    n_vt = vocab // 128
    tail = vocab - n_vt * 128
    per_w = (n_vt + _NW - 1) // _NW
    mesh = plsc.VectorSubcoreMesh(core_axis_name="c", subcore_axis_name="s")

    @functools.partial(
        pl.kernel,
        mesh=mesh,
        out_type=jax.ShapeDtypeStruct((vocab * emb,), jnp.float32),
        scratch_types=[
            pltpu.VMEM((64, 128), jnp.float32),
            pltpu.VMEM((3 * 4096,), jnp.float32),
            pltpu.VMEM((tail * emb,), jnp.float32),
            pltpu.SemaphoreType.DMA((2,)),
            pltpu.SemaphoreType.DMA((3,)),
        ],
        compiler_params=pltpu.CompilerParams(
            use_tc_tiling_on_sc=True, needs_layout_passes=False
        ),
    )
    def detile_kernel(table_t, tail_rm, tr_out, vin, vout, vtail, lsem, osem):
        wid = lax.axis_index("s") * 2 + lax.axis_index("c")
        lanes = lax.iota(jnp.int32, _LANE)

        def fire_loads(vt, p):
            for eh in range(4):
                pltpu.async_copy(
                    table_t.at[pl.ds(eh * 8, 8), pl.ds(vt * 128, 128)],
                    vin.at[pl.ds(p * 32 + eh * 8, 8)],
                    lsem.at[p],
                )

        def wait_loads(p):
            pltpu.make_async_copy(
                table_t.at[pl.ds(0, 32), pl.ds(0, 128)],
                vin.at[pl.ds(p * 32, 32)],
                lsem.at[p],
            ).wait()

        def wait_out(s):
            pltpu.make_async_copy(
                vout.at[pl.ds(s * 4096, 4096)],
                tr_out.at[pl.ds(0, 4096)],
                osem.at[s],
            ).wait()

        def transpose_and_write(vt, p, s):
            row0 = lanes + p * 32
            row1 = row0 + 16
            so = s * 4096

            @plsc.parallel_loop(0, 128, unroll=8)
            def _(vl):
                vlv = lanes * 0 + vl
                vout[pl.ds(so + vl * 32, _LANE)] = plsc.load_gather(
                    vin, [row0, vlv]
                )
                vout[pl.ds(so + vl * 32 + 16, _LANE)] = plsc.load_gather(
                    vin, [row1, vlv]
                )

            pltpu.async_copy(
                vout.at[pl.ds(s * 4096, 4096)],
                tr_out.at[pl.ds(vt * 4096, 4096)],
                osem.at[s],
            )

        fire_loads(wid, 0)

        @pl.loop(0, per_w)
        def _(i):
            vt = i * _NW + wid
            vt_next = vt + _NW
            p = lax.rem(i, 2)
            s = lax.rem(i, 3)

            @pl.when(vt_next < n_vt)
            def _():
                fire_loads(vt_next, 1 - p)

            @pl.when(vt < n_vt)
            def _():
                wait_loads(p)

                @pl.when(i >= 3)
                def _():
                    wait_out(s)

                transpose_and_write(vt, p, s)

        for d in range(max(0, per_w - 4), per_w):
            vt_d = d * _NW + wid

            @pl.when((vt_d < n_vt) & (vt_d + 3 * _NW >= n_vt))
            def _():
                wait_out(d % 3)

        @pl.when(wid == 0)
        def _():
            pltpu.sync_copy(tail_rm, vtail)
            pltpu.sync_copy(vtail, tr_out.at[pl.ds(n_vt * 4096, tail * emb)])

    return detile_kernel


def _make_gather(n_rows, vocab, emb):
    rows_per_w = n_rows // _NW
    chunk = 8
    n_chunks = rows_per_w // chunk
    unit = 128 * emb
    mesh = plsc.VectorSubcoreMesh(core_axis_name="c", subcore_axis_name="s")

    @functools.partial(
        pl.kernel,
        mesh=mesh,
        out_type=jax.ShapeDtypeStruct((200 * 131072,), jnp.float32),
        scratch_types=[
            pltpu.VMEM((2 * chunk, 128), jnp.int32),
            pltpu.VMEM((2 * chunk * 128, emb), jnp.float32),
            pltpu.VMEM((4 * unit,), jnp.float32),
            pltpu.SemaphoreType.DMA((2,)),
            pltpu.SemaphoreType.DMA((4,)),
        ],
        compiler_params=pltpu.CompilerParams(
            use_tc_tiling_on_sc=False, needs_layout_passes=False
        ),
    )
    def gather_kernel(idx_hbm, table_rm, out_hbm, idx_v, rows_v, stg, gsem, osem):
        wid = lax.axis_index("s") * 2 + lax.axis_index("c")
        base_row = wid * rows_per_w
        lanes = lax.iota(jnp.int32, _LANE)

        def fire_chunk(c, p):
            r = base_row + c * chunk
            pltpu.sync_copy(
                idx_hbm.at[pl.ds(r, chunk)], idx_v.at[pl.ds(p * chunk, chunk)]
            )
            for j in range(chunk):
                pltpu.async_copy(
                    table_rm.at[idx_v.at[p * chunk + j]],
                    rows_v.at[pl.ds((p * chunk + j) * 128, 128)],
                    gsem.at[p],
                )

        def wait_gathers(p):
            pltpu.make_async_copy(
                table_rm.at[pl.ds(0, chunk * 128)],
                rows_v.at[pl.ds(p * chunk * 128, chunk * 128)],
                gsem.at[p],
            ).wait()

        def emit_chunk(ec, p):
            r = base_row + ec * chunk

            @pl.loop(0, chunk)
            def _(j):
                rr = r + j
                l = lax.div(rr, 32)
                bj = lax.rem(rr, 32)
                gu = ec * chunk + j
                sp = lax.rem(gu, 4)

                @pl.when(gu >= 4)
                def _():
                    pltpu.make_async_copy(
                        stg.at[pl.ds(0, unit)],
                        out_hbm.at[pl.ds(0, unit)],
                        osem.at[sp],
                    ).wait()

                src0 = (p * chunk + j) * 128
                sbase = sp * unit
                rowv = [lanes + (src0 + b0) for b0 in range(0, 128, _LANE)]

                @plsc.parallel_loop(0, emb, unroll=4)
                def _(e):
                    ev = lanes * 0 + e
                    for bi in range(8):
                        vals = plsc.load_gather(rows_v, [rowv[bi], ev])
                        stg[pl.ds(sbase + e * 128 + bi * _LANE, _LANE)] = vals

                for eh in range(4):
                    pltpu.async_copy(
                        stg.at[pl.ds(sp * unit + eh * 1024, 1024)],
                        out_hbm.at[
                            pl.ds(l * 131072 + eh * 32 * 1024 + bj * 1024, 1024)
                        ],
                        osem.at[sp],
                    )

        fire_chunk(0, 0)

        @pl.loop(0, (n_chunks - 1) // 2)
        def _(g):
            for t in range(2):
                c = 1 + g * 2 + t
                p = (1 + t) % 2
                fire_chunk(c, p)
                wait_gathers(1 - p)
                emit_chunk(c - 1, 1 - p)

        p_last = (n_chunks - 1) % 2
        wait_gathers(p_last)
        emit_chunk(n_chunks - 1, p_last)
        for sp in range(4):
            pltpu.make_async_copy(
                stg.at[pl.ds(0, unit)], out_hbm.at[pl.ds(0, unit)], osem.at[sp]
            ).wait()

    return gather_kernel


def kernel(x, table):
    b, l = x.shape
    vocab, emb = table.shape
    n = b * l

    table_t = table.T
    n_vt = vocab // 128
    tail_rm = table[n_vt * 128:, :].reshape(-1)
    tr = _make_detile(vocab, emb)(table_t, tail_rm)
    table_rm = tr.reshape(vocab, emb)

    idx = x.T.reshape(n // 128, 128)
    out5 = _make_gather(n // 128, vocab, emb)(idx, table_rm)

    out = (
        out5.reshape(l, 4, 32, 8, 128)
        .transpose(2, 4, 0, 1, 3)
        .reshape(b, l, emb)
    )
    return out
